# baseline (speedup 1.0000x reference)
import functools

import jax
import jax.numpy as jnp
from jax import lax
from jax.experimental import pallas as pl
from jax.experimental.pallas import tpu as pltpu
from jax.experimental.pallas import tpu_sc as plsc

N = 50000
NH = 25000
V = 5143
W = 8
D = 32
K = 9
NG = 41
VP = NG * 128
R = 512
NHP = 25088
NPAD = 50176
NKH = NH * K
NMC = 220
NKHP = NMC * 1024
NSBH = NKHP // 128
TOPC = 4
PAD_COORD = 1e4


def _knn_body(c0t_ref, m_ref, idx_ref):
    c0t = c0t_ref[...]
    vsq = jnp.sum(c0t * c0t, axis=0, keepdims=True)
    m = m_ref[...]
    msq = jnp.sum(m * m, axis=1, keepdims=True)
    mm = lax.dot_general(m, c0t, (((1,), (0,)), ((), ())),
                         preferred_element_type=jnp.float32)
    d = msq - 2.0 * mm + vsq
    INF = jnp.float32(jnp.inf)
    BIG = jnp.float32(1e9)

    planes = [d[:, j * 128:(j + 1) * 128] for j in range(NG)]
    lane = lax.broadcasted_iota(jnp.int32, (1, 128), 1).astype(jnp.float32)

    def tree_lexmin(items):
        while len(items) > 1:
            nxt = []
            for a, b in zip(items[0::2], items[1::2]):
                take_a = a[0] <= b[0]
                nxt.append((jnp.where(take_a, a[0], b[0]),
                            jnp.where(take_a, a[1], b[1])))
            if len(items) % 2:
                nxt.append(items[-1])
            items = nxt
        return items[0]

    cand_v, cand_i = [], []
    for r in range(TOPC):
        mv, wv = tree_lexmin([(p, jnp.float32(j))
                              for j, p in enumerate(planes)])
        cand_v.append(mv)
        cand_i.append(wv * 128.0 + lane)
        if r < TOPC - 1:
            planes = [jnp.where(wv == jnp.float32(j), INF, p)
                      for j, p in enumerate(planes)]

    idxs = []
    for _ in range(K):
        m4 = functools.reduce(jnp.minimum, cand_v)
        mn = jnp.min(m4, axis=1, keepdims=True)
        i4 = functools.reduce(jnp.minimum, [
            jnp.where(v == mn, i, BIG) for v, i in zip(cand_v, cand_i)])
        am = jnp.min(i4, axis=1, keepdims=True)
        idxs.append(am)
        cand_v = [jnp.where(i == am, INF, v) for v, i in zip(cand_v, cand_i)]
    idx_ref[...] = jnp.concatenate(idxs, axis=1).astype(jnp.int32)


_knn_call = pl.pallas_call(
    _knn_body,
    grid=(NHP // R,),
    in_specs=[
        pl.BlockSpec((8, VP), lambda i: (0, 0)),
        pl.BlockSpec((R, 8), lambda i: (i, 0)),
    ],
    out_specs=pl.BlockSpec((R, K), lambda i: (i, 0)),
    out_shape=jax.ShapeDtypeStruct((NHP, K), jnp.int32),
)


def _sqrt_body(x_ref, o_ref):
    o_ref[...] = jnp.sqrt(x_ref[...] + 1e-12)


_sqrt_call = pl.pallas_call(
    _sqrt_body,
    grid=(2,),
    in_specs=[pl.BlockSpec((NSBH // 2, 128), lambda i: (i, 0))],
    out_specs=pl.BlockSpec((NSBH // 2, 128), lambda i: (i, 0)),
    out_shape=jax.ShapeDtypeStruct((NSBH, 128), jnp.float32),
)


def _emit_macro(mc, goff, idx_hbm, ql_hbm, mot_hbm, lat_hbm, c8_hbm,
                means_hbm, out_mot, out_lat, out_d2,
                idx_v, ql_v, mot_v, lat_v, c8_v, mns_v, d2_v,
                sem_m, sem_l, sem_c):
    sb0 = pl.multiple_of(mc * 8, 8)
    base = pl.multiple_of(mc * 1024, 1024)
    pltpu.sync_copy(idx_hbm.at[pl.ds(sb0, 8)], idx_v)
    pltpu.sync_copy(ql_hbm.at[pl.ds(base, 1024)], ql_v)
    q0 = pl.multiple_of(((goff + base) // K) & ~7, 8)
    pltpu.sync_copy(means_hbm.at[pl.ds(q0, 128)], mns_v)
    cps = []
    for j in range(8):
        cps.append(pltpu.async_copy(
            mot_hbm.at[idx_v.at[j]], mot_v.at[pl.ds(j * 128, 128)], sem_m))
        cps.append(pltpu.async_copy(
            lat_hbm.at[idx_v.at[j]], lat_v.at[pl.ds(j * 128, 128)], sem_l))
        cps.append(pltpu.async_copy(
            c8_hbm.at[idx_v.at[j]], c8_v.at[pl.ds(j * 128, 128)], sem_c))
    for cp in cps:
        cp.wait()

    def dist_body(t, carry):
        rloc = t * 16 + lax.iota(jnp.int32, 16)
        qloc = ql_v[pl.ds(t * 16, 16)]
        acc = jnp.zeros((16,), jnp.float32)
        for c in range(8):
            cvec = jnp.full((16,), c, jnp.int32)
            cc = plsc.load_gather(c8_v, [rloc, cvec])
            mmc = plsc.load_gather(mns_v, [qloc, cvec])
            df = mmc - cc
            acc = acc + df * df
        d2_v[pl.ds(t * 16, 16)] = acc
        return carry

    lax.fori_loop(0, 64, dist_body, 0)

    pltpu.sync_copy(mot_v, out_mot.at[pl.ds(base, 1024)])
    pltpu.sync_copy(lat_v, out_lat.at[pl.ds(base, 1024)])
    pltpu.sync_copy(d2_v, out_d2.at[pl.ds(base, 1024)])


def _make_gather(goff):
    info = plsc.get_sparse_core_info()
    nc = info.num_cores
    nw = nc * info.num_subcores

    mesh = plsc.VectorSubcoreMesh(core_axis_name="c", subcore_axis_name="s")

    @functools.partial(
        pl.kernel, mesh=mesh,
        compiler_params=pltpu.CompilerParams(use_tc_tiling_on_sc=False,
                                             needs_layout_passes=False),
        out_type=[
            jax.ShapeDtypeStruct((NKHP, W * 3), jnp.float32),
            jax.ShapeDtypeStruct((NKHP, D), jnp.float32),
            jax.ShapeDtypeStruct((NKHP,), jnp.float32),
        ],
        scratch_types=[
            pltpu.VMEM((8, 128), jnp.int32),
            pltpu.VMEM((1024,), jnp.int32),
            pltpu.VMEM((1024, W * 3), jnp.float32),
            pltpu.VMEM((1024, D), jnp.float32),
            pltpu.VMEM((1024, 8), jnp.float32),
            pltpu.VMEM((128, 8), jnp.float32),
            pltpu.VMEM((1024,), jnp.float32),
            pltpu.SemaphoreType.DMA,
            pltpu.SemaphoreType.DMA,
            pltpu.SemaphoreType.DMA,
        ],
    )
    def gather(idx_hbm, ql_hbm, mot_hbm, lat_hbm, c8_hbm, means_hbm,
               out_mot, out_lat, out_d2,
               idx_v, ql_v, mot_v, lat_v, c8_v, mns_v, d2_v,
               sem_m, sem_l, sem_c):
        wid = lax.axis_index("s") * nc + lax.axis_index("c")

        def body(i, carry):
            mc = i * nw + wid

            @pl.when(mc < NMC)
            def _():
                _emit_macro(mc, goff, idx_hbm, ql_hbm, mot_hbm, lat_hbm,
                            c8_hbm, means_hbm, out_mot, out_lat, out_d2,
                            idx_v, ql_v, mot_v, lat_v, c8_v, mns_v, d2_v,
                            sem_m, sem_l, sem_c)

            return carry

        lax.fori_loop(0, (NMC + nw - 1) // nw, body, 0)

    return gather


def _half_qloc(goff):
    r = jnp.arange(NKHP, dtype=jnp.int32)
    return (goff + r) // K - ((goff + (r // 1024) * 1024) // K & ~7)


def kernel(means, vertex_positions, canonical_vertex_positions, latents_table):
    c0 = canonical_vertex_positions[0]
    vm = vertex_positions - canonical_vertex_positions
    motion = jnp.transpose(vm, (1, 0, 2)).reshape(V, W * 3)
    c8 = jnp.zeros((V, 8), jnp.float32).at[:, :3].set(c0)

    c0t = jnp.zeros((8, VP), jnp.float32)
    c0t = c0t.at[:3, :V].set(c0.T)
    c0t = c0t.at[:3, V:].set(PAD_COORD)
    means_p = jnp.zeros((NPAD, 8), jnp.float32).at[:N, :3].set(means)

    halves = []
    for h in (0, 1):
        goff = h * NKH
        means_h = lax.dynamic_slice(means_p, (h * NH, 0), (NHP, 8))
        knn_idx = _knn_call(c0t, means_h)
        idx_flat = knn_idx[:NH].reshape(NKH)
        idx2d = jnp.pad(idx_flat, (0, NKHP - NKH)).reshape(NSBH, 128)
        out_mot, out_lat, d2 = _make_gather(goff)(
            idx2d, _half_qloc(goff), motion, latents_table, c8, means_p)
        dists = _sqrt_call(d2.reshape(NSBH, 128)).reshape(NKHP)
        halves.append((out_mot[:NKH], out_lat[:NKH], dists[:NKH]))

    out_mot = jnp.concatenate([halves[0][0], halves[1][0]])
    out_lat = jnp.concatenate([halves[0][1], halves[1][1]])
    dists = jnp.concatenate([halves[0][2], halves[1][2]])
    return (out_mot.reshape(N, K, W * 3),
            out_lat.reshape(N, K, D),
            dists.reshape(N, K, 1))


# speedup vs baseline: 1.5739x; 1.5739x over previous
import functools

import jax
import jax.numpy as jnp
from jax import lax
from jax.experimental import pallas as pl
from jax.experimental.pallas import tpu as pltpu
from jax.experimental.pallas import tpu_sc as plsc

N = 50000
V = 5143
W = 8
D = 32
K = 9
NG = 41
VP = NG * 128
R = 512
NPAD = 50176
NK = N * K
NKP = 450048
NSB = NKP // 128
NMC = 440
TOPC = 4
PAD_COORD = 1e4


def _knn_body(c0t_ref, m_ref, idx_ref):
    c0tn = c0t_ref[...]
    vsq = 0.25 * jnp.sum(c0tn * c0tn, axis=0, keepdims=True)
    m = m_ref[...]
    msq = jnp.sum(m * m, axis=1, keepdims=True)
    mm = lax.dot_general(m, c0tn, (((1,), (0,)), ((), ())),
                         preferred_element_type=jnp.float32)
    d = msq + mm + vsq
    INF = jnp.float32(jnp.inf)
    BIG = jnp.float32(1e9)

    planes = [d[:, j * 128:(j + 1) * 128] for j in range(NG)]
    lane = lax.broadcasted_iota(jnp.int32, (1, 128), 1).astype(jnp.float32)

    def tree_lexmin(items):
        while len(items) > 1:
            nxt = []
            for a, b in zip(items[0::2], items[1::2]):
                take_a = a[0] <= b[0]
                nxt.append((jnp.where(take_a, a[0], b[0]),
                            jnp.where(take_a, a[1], b[1])))
            if len(items) % 2:
                nxt.append(items[-1])
            items = nxt
        return items[0]

    cand_v, cand_i = [], []
    for r in range(TOPC):
        mv, wv = tree_lexmin([(p, jnp.float32(j))
                              for j, p in enumerate(planes)])
        cand_v.append(mv)
        cand_i.append(wv * 128.0 + lane)
        if r < TOPC - 1:
            planes = [jnp.where(wv == jnp.float32(j), INF, p)
                      for j, p in enumerate(planes)]

    idxs = []
    for _ in range(K):
        m4 = functools.reduce(jnp.minimum, cand_v)
        mn = jnp.min(m4, axis=1, keepdims=True)
        i4 = functools.reduce(jnp.minimum, [
            jnp.where(v == mn, i, BIG) for v, i in zip(cand_v, cand_i)])
        am = jnp.min(i4, axis=1, keepdims=True)
        idxs.append(am)
        cand_v = [jnp.where(i == am, INF, v) for v, i in zip(cand_v, cand_i)]
    idx_ref[...] = jnp.concatenate(idxs, axis=1).astype(jnp.int32)


_knn_call = pl.pallas_call(
    _knn_body,
    grid=(NPAD // R,),
    in_specs=[
        pl.BlockSpec((8, VP), lambda i: (0, 0)),
        pl.BlockSpec((R, 8), lambda i: (i, 0)),
    ],
    out_specs=pl.BlockSpec((R, K), lambda i: (i, 0)),
    out_shape=jax.ShapeDtypeStruct((NPAD, K), jnp.int32),
)


def _sqrt_body(x_ref, o_ref):
    o_ref[...] = jnp.sqrt(x_ref[...] + 1e-12)


_sqrt_call = pl.pallas_call(
    _sqrt_body,
    grid=(1,),
    in_specs=[pl.BlockSpec((NSB, 128), lambda i: (0, 0))],
    out_specs=pl.BlockSpec((NSB, 128), lambda i: (0, 0)),
    out_shape=jax.ShapeDtypeStruct((NSB, 128), jnp.float32),
)


def _emit_macro(mc, nsb, wrows, idx_hbm, ql_hbm, mot_hbm, lat_hbm, c8_hbm,
                means_hbm, out_mot, out_lat, out_d2,
                idx_v, ql_v, mot_v, lat_v, c8_v, mns_v, d2_v,
                sem_m, sem_l, sem_c):
    sb0 = pl.multiple_of(mc * 8, 8)
    base = pl.multiple_of(mc * 1024, 1024)
    pltpu.sync_copy(idx_hbm.at[pl.ds(sb0, nsb)], idx_v.at[pl.ds(0, nsb)])
    pltpu.sync_copy(ql_hbm.at[pl.ds(base, 1024)], ql_v)
    q0 = pl.multiple_of((base // K) & ~7, 8)
    pltpu.sync_copy(means_hbm.at[pl.ds(q0, 128)], mns_v)
    cps = []
    for j in range(nsb):
        cps.append(pltpu.async_copy(
            mot_hbm.at[idx_v.at[j]], mot_v.at[pl.ds(j * 128, 128)], sem_m))
        cps.append(pltpu.async_copy(
            lat_hbm.at[idx_v.at[j]], lat_v.at[pl.ds(j * 128, 128)], sem_l))
        cps.append(pltpu.async_copy(
            c8_hbm.at[idx_v.at[j]], c8_v.at[pl.ds(j * 128, 128)], sem_c))
    for cp in cps:
        cp.wait()

    def dist_body(t, carry):
        rloc = t * 16 + lax.iota(jnp.int32, 16)
        qloc = ql_v[pl.ds(t * 16, 16)]
        acc = jnp.zeros((16,), jnp.float32)
        for c in range(8):
            cvec = jnp.full((16,), c, jnp.int32)
            cc = plsc.load_gather(c8_v, [rloc, cvec])
            mmc = plsc.load_gather(mns_v, [qloc, cvec])
            df = mmc - cc
            acc = acc + df * df
        d2_v[pl.ds(t * 16, 16)] = acc
        return carry

    lax.fori_loop(0, wrows // 16, dist_body, 0)

    pltpu.sync_copy(mot_v.at[pl.ds(0, wrows)], out_mot.at[pl.ds(base, wrows)])
    pltpu.sync_copy(lat_v.at[pl.ds(0, wrows)], out_lat.at[pl.ds(base, wrows)])
    pltpu.sync_copy(d2_v.at[pl.ds(0, wrows)], out_d2.at[pl.ds(base, wrows)])


def _make_gather():
    info = plsc.get_sparse_core_info()
    nc = info.num_cores
    nw = nc * info.num_subcores

    mesh = plsc.VectorSubcoreMesh(core_axis_name="c", subcore_axis_name="s")

    @functools.partial(
        pl.kernel, mesh=mesh,
        compiler_params=pltpu.CompilerParams(use_tc_tiling_on_sc=False,
                                             needs_layout_passes=False),
        out_type=[
            jax.ShapeDtypeStruct((NK, W * 3), jnp.float32),
            jax.ShapeDtypeStruct((NK, D), jnp.float32),
            jax.ShapeDtypeStruct((NKP,), jnp.float32),
        ],
        scratch_types=[
            pltpu.VMEM((8, 128), jnp.int32),
            pltpu.VMEM((1024,), jnp.int32),
            pltpu.VMEM((1024, W * 3), jnp.float32),
            pltpu.VMEM((1024, D), jnp.float32),
            pltpu.VMEM((1024, 8), jnp.float32),
            pltpu.VMEM((128, 8), jnp.float32),
            pltpu.VMEM((1024,), jnp.float32),
            pltpu.SemaphoreType.DMA,
            pltpu.SemaphoreType.DMA,
            pltpu.SemaphoreType.DMA,
        ],
    )
    def gather(idx_hbm, ql_hbm, mot_hbm, lat_hbm, c8_hbm, means_hbm,
               out_mot, out_lat, out_d2,
               idx_v, ql_v, mot_v, lat_v, c8_v, mns_v, d2_v,
               sem_m, sem_l, sem_c):
        wid = lax.axis_index("s") * nc + lax.axis_index("c")

        def body(i, carry):
            mc = i * nw + wid
            args = (idx_hbm, ql_hbm, mot_hbm, lat_hbm, c8_hbm, means_hbm,
                    out_mot, out_lat, out_d2,
                    idx_v, ql_v, mot_v, lat_v, c8_v, mns_v, d2_v,
                    sem_m, sem_l, sem_c)

            @pl.when(mc < NMC - 1)
            def _():
                _emit_macro(mc, 8, 1024, *args)

            @pl.when(mc == NMC - 1)
            def _():
                _emit_macro(mc, 4, NK - (NMC - 1) * 1024, *args)

            return carry

        lax.fori_loop(0, (NMC + nw - 1) // nw, body, 0)

    return gather


def kernel(means, vertex_positions, canonical_vertex_positions, latents_table):
    c0 = canonical_vertex_positions[0]
    vm = vertex_positions - canonical_vertex_positions
    motion = jnp.transpose(vm, (1, 0, 2)).reshape(V, W * 3)
    c8 = jnp.zeros((V, 8), jnp.float32).at[:, :3].set(c0)

    c0t = jnp.zeros((8, VP), jnp.float32)
    c0t = c0t.at[:3, :V].set(c0.T)
    c0t = c0t.at[:3, V:].set(PAD_COORD)
    c0t = -2.0 * c0t
    means_p = jnp.zeros((NPAD, 8), jnp.float32).at[:N, :3].set(means)

    knn_idx = _knn_call(c0t, means_p)

    idx_flat = knn_idx[:N].reshape(NK)
    idx2d = jnp.pad(idx_flat, (0, NKP - NK)).reshape(NSB, 128)
    rows = jnp.arange(NKP, dtype=jnp.int32)
    qloc = rows // K - (((rows // 1024) * 1024) // K & ~7)

    out_mot, out_lat, d2 = _make_gather()(idx2d, qloc, motion, latents_table,
                                          c8, means_p)
    dists = _sqrt_call(d2.reshape(NSB, 128)).reshape(NKP)[:NK]

    return (out_mot.reshape(N, K, W * 3),
            out_lat.reshape(N, K, D),
            dists.reshape(N, K, 1))
